# baseline (speedup 1.0000x reference)
import functools

import jax
import jax.numpy as jnp
from jax import lax
from jax.experimental import pallas as pl
from jax.experimental.pallas import tpu as pltpu
from jax.experimental.pallas import tpu_sc as plsc

N = 10000
E = 320000
IN_DIM = 128
EPS = 1e-5

NC, NS = 2, 16
NW = NC * NS
CH = 128
NCH = 80
EP = NW * NCH * CH
NPAD = 10240
RPT = NPAD // NS
ZB = 128

_MESH = plsc.VectorSubcoreMesh(
    core_axis_name="c", subcore_axis_name="s", num_cores=NC, num_subcores=NS
)


def _fill(buf, nrows, width, value):
    @pl.loop(0, nrows)
    def _(i):
        for j in range(width // 16):
            buf[i, pl.ds(j * 16, 16)] = jnp.full((16,), value, jnp.float32)


def _zero_acc_rows(zbuf, acc, r0, zb=ZB):
    for k in range(RPT // zb):
        pltpu.sync_copy(zbuf, acc.at[pl.ds(r0 + k * zb, zb)])


def _dump_acc(acc, out_hbm, c, r0):
    @pl.when(c == 0)
    def _():
        pltpu.sync_copy(acc.at[pl.ds(r0, RPT)], out_hbm.at[0, pl.ds(r0, RPT)])

    @pl.when(c == 1)
    def _():
        pltpu.sync_copy(acc.at[pl.ds(r0, RPT)], out_hbm.at[1, pl.ds(r0, RPT)])


D64 = 64
MBUF = 3
LOOK = 2
NML = NCH - 2


def _agg_pass(zp_hbm, out_hbm, acc, zps, src_all, dst_all, b0, b1, b2,
              gsem, ssem, c, s, r0):
    bufs = (b0, b1, b2)
    _fill(bufs[0], CH, D64, 0.0)
    _zero_acc_rows(bufs[0], acc, r0, CH)
    @pl.when(s < NS - 1)
    def _():
        pltpu.sync_copy(zp_hbm.at[pl.ds(r0, RPT)], zps.at[pl.ds(r0, RPT)])

    @pl.when(s == NS - 1)
    def _():
        pltpu.sync_copy(zp_hbm.at[pl.ds((NS - 1) * RPT, N - (NS - 1) * RPT)],
                        zps.at[pl.ds((NS - 1) * RPT, N - (NS - 1) * RPT)])

    plsc.subcore_barrier()

    for m in range(LOOK):
        pltpu.async_copy(zps.at[src_all.at[m]], bufs[m], gsem)

    @pl.loop(0, NML, step=MBUF)
    def _(j):
        for m in range(MBUF):
            jj = j + m
            b = bufs[m]
            bn = bufs[(m + LOOK) % MBUF]
            pltpu.make_async_copy(zps.at[src_all.at[jj]], b, gsem).wait()
            pltpu.async_copy(b, acc.at[dst_all.at[jj]], ssem, add=True)

            @pl.when(jj >= 1)
            def _():
                pltpu.make_async_copy(bn, acc.at[dst_all.at[jj - 1]], ssem).wait()

            @pl.when(jj + LOOK < NCH)
            def _():
                pltpu.async_copy(zps.at[src_all.at[jj + LOOK]], bn, gsem)

    for jj in (NML, NML + 1):
        b = bufs[jj % MBUF]
        pltpu.make_async_copy(zps.at[src_all.at[jj]], b, gsem).wait()
        pltpu.async_copy(b, acc.at[dst_all.at[jj]], ssem, add=True)
    for jj in (NML - 1, NML, NML + 1):
        pltpu.make_async_copy(bufs[jj % MBUF], acc.at[dst_all.at[jj]], ssem).wait()

    plsc.subcore_barrier()
    _dump_acc(acc, out_hbm, c, r0)


_AGG_SCRATCH = [
    pltpu.VMEM_SHARED((NPAD, D64), jnp.float32),
    pltpu.VMEM_SHARED((NPAD, D64), jnp.float32),
    pltpu.VMEM((NCH, CH), jnp.int32),
    pltpu.VMEM((NCH, CH), jnp.int32),
    [pltpu.VMEM((CH, D64), jnp.float32) for _ in range(MBUF)],
    pltpu.SemaphoreType.DMA,
    pltpu.SemaphoreType.DMA,
    pltpu.SemaphoreType.DMA,
]


def _agg_prelude(src_hbm, dst_hbm, src_all, dst_all, isem):
    c = lax.axis_index("c")
    s = lax.axis_index("s")
    wid = s * NC + c
    r0 = s * RPT
    pltpu.async_copy(src_hbm.at[wid], src_all, isem)
    pltpu.async_copy(dst_hbm.at[wid], dst_all, isem)
    pltpu.make_async_copy(src_hbm.at[wid], src_all, isem).wait()
    pltpu.make_async_copy(dst_hbm.at[wid], dst_all, isem).wait()
    return c, s, r0


@functools.partial(
    pl.kernel,
    out_type=jax.ShapeDtypeStruct((NC, NPAD, D64), jnp.float32),
    mesh=_MESH,
    scratch_types=_AGG_SCRATCH,
    compiler_params=pltpu.CompilerParams(use_tc_tiling_on_sc=False),
)
def _agg64(zp_hbm, src_hbm, dst_hbm, out_hbm, acc, zps, src_all, dst_all,
           bufs, isem, gsem, ssem):
    c, s, r0 = _agg_prelude(src_hbm, dst_hbm, src_all, dst_all, isem)
    _agg_pass(zp_hbm, out_hbm, acc, zps, src_all, dst_all, bufs[0], bufs[1], bufs[2],
              gsem, ssem, c, s, r0)


@functools.partial(
    pl.kernel,
    out_type=[
        jax.ShapeDtypeStruct((NC, NPAD, D64), jnp.float32),
        jax.ShapeDtypeStruct((NC, NPAD, D64), jnp.float32),
    ],
    mesh=_MESH,
    scratch_types=_AGG_SCRATCH,
    compiler_params=pltpu.CompilerParams(use_tc_tiling_on_sc=False),
)
def _agg64x2(zlo_hbm, zhi_hbm, src_hbm, dst_hbm, outlo_hbm, outhi_hbm,
             acc, zps, src_all, dst_all, bufs, isem, gsem, ssem):
    c, s, r0 = _agg_prelude(src_hbm, dst_hbm, src_all, dst_all, isem)
    _agg_pass(zlo_hbm, outlo_hbm, acc, zps, src_all, dst_all, bufs[0], bufs[1], bufs[2],
              gsem, ssem, c, s, r0)
    _agg_pass(zhi_hbm, outhi_hbm, acc, zps, src_all, dst_all, bufs[0], bufs[1], bufs[2],
              gsem, ssem, c, s, r0)

DW = 16


@functools.partial(
    pl.kernel,
    out_type=jax.ShapeDtypeStruct((NC, NPAD, DW), jnp.float32),
    mesh=_MESH,
    scratch_types=[
        pltpu.VMEM_SHARED((NPAD, DW), jnp.float32),
        pltpu.VMEM((NCH, CH), jnp.int32),
        pltpu.VMEM((ZB, DW), jnp.float32),
        pltpu.VMEM((CH, DW), jnp.float32),
    ],
    compiler_params=pltpu.CompilerParams(use_tc_tiling_on_sc=False),
)
def _deg_kernel(dst_hbm, out_hbm, acc, dst_all, zbuf, ones_v):
    c = lax.axis_index("c")
    s = lax.axis_index("s")
    wid = s * NC + c
    r0 = s * RPT
    _fill(zbuf, ZB, DW, 0.0)
    _fill(ones_v, CH, DW, 1.0)
    _zero_acc_rows(zbuf, acc, r0)
    pltpu.sync_copy(dst_hbm.at[wid], dst_all)
    plsc.subcore_barrier()

    @pl.loop(0, NCH)
    def _(j):
        pltpu.sync_copy(ones_v, acc.at[dst_all.at[j]], add=True)

    plsc.subcore_barrier()
    _dump_acc(acc, out_hbm, c, r0)


BLK = 512
GRID = (N + BLK - 1) // BLK


def _row_spec(d):
    return pl.BlockSpec((BLK, d), lambda i: (i, 0))


def _full_spec(shape):
    nd = len(shape)
    return pl.BlockSpec(shape, lambda i: (0,) * nd)


def _part_spec(d):
    return pl.BlockSpec((2, BLK, d), lambda i: (0, i, 0))


def _layer_norm(pre, g, b):
    mu = jnp.mean(pre, axis=-1, keepdims=True)
    var = jnp.mean((pre - mu) ** 2, axis=-1, keepdims=True)
    return (pre - mu) / jnp.sqrt(var + EPS) * g + b


def _tc_a(x, W0, degp):
    def body(x_ref, w_ref, dp_ref, z_ref, dv_ref):
        dp = dp_ref[...]
        deg = dp[0, :, 0] + dp[1, :, 0] + 1.0
        dv = lax.rsqrt(deg)[:, None]
        dv_ref[...] = dv
        z_ref[...] = jnp.dot(x_ref[...], w_ref[...], preferred_element_type=jnp.float32) * dv

    return pl.pallas_call(
        body,
        grid=(GRID,),
        in_specs=[_row_spec(IN_DIM), _full_spec((IN_DIM, 64)), _part_spec(DW)],
        out_specs=[_row_spec(64), _row_spec(1)],
        out_shape=[
            jax.ShapeDtypeStruct((N, 64), jnp.float32),
            jax.ShapeDtypeStruct((N, 1), jnp.float32),
        ],
    )(x, W0, degp)


def _tc_b0(s0, z0p, dinv, b0, g0, beta0):
    def body(sp_ref, zp_ref, dv_ref, b_ref, g_ref, be_ref, out_ref):
        sp = sp_ref[...]
        dv = dv_ref[...]
        pre = (sp[0] + sp[1] + zp_ref[...]) * dv + b_ref[...]
        h = jnp.maximum(_layer_norm(pre, g_ref[...], be_ref[...]), 0.0)
        out_ref[...] = h * dv

    return pl.pallas_call(
        body,
        grid=(GRID,),
        in_specs=[
            _part_spec(64), _row_spec(64), _row_spec(1),
            _full_spec((1, 64)), _full_spec((1, 64)), _full_spec((1, 64)),
        ],
        out_specs=_row_spec(64),
        out_shape=jax.ShapeDtypeStruct((N, 64), jnp.float32),
    )(s0, z0p, dinv, b0, g0, beta0)


def _tc_b1(s1, z1p, dinv, W1, b1, g1, beta1, W2):
    def body(sp_ref, zp_ref, dv_ref, w1_ref, b_ref, g_ref, be_ref, w2_ref,
             lo_ref, hi_ref):
        sp = sp_ref[...]
        dv = dv_ref[...]
        agg = (sp[0] + sp[1] + zp_ref[...]) * dv
        pre = jnp.dot(agg, w1_ref[...], preferred_element_type=jnp.float32) + b_ref[...]
        h = jnp.maximum(_layer_norm(pre, g_ref[...], be_ref[...]), 0.0)
        z2 = jnp.dot(h, w2_ref[...], preferred_element_type=jnp.float32) * dv
        lo_ref[...] = z2[:, :64]
        hi_ref[...] = z2[:, 64:]

    return pl.pallas_call(
        body,
        grid=(GRID,),
        in_specs=[
            _part_spec(64), _row_spec(64), _row_spec(1),
            _full_spec((64, 128)), _full_spec((1, 128)), _full_spec((1, 128)),
            _full_spec((1, 128)), _full_spec((128, 128)),
        ],
        out_specs=[_row_spec(64), _row_spec(64)],
        out_shape=[
            jax.ShapeDtypeStruct((N, 64), jnp.float32),
            jax.ShapeDtypeStruct((N, 64), jnp.float32),
        ],
    )(s1, z1p, dinv, W1, b1, g1, beta1, W2)


def _tc_b2(s2lo, s2hi, z2lo, z2hi, dinv, b2, g2, beta2, Wm1, bm1, Wm2, bm2, Wm3, bm3):
    def body(slo_ref, shi_ref, zlo_ref, zhi_ref, dv_ref, b_ref, g_ref, be_ref,
             w1_ref, c1_ref, w2_ref, c2_ref, w3_ref, c3_ref, out_ref):
        slo = slo_ref[...]
        shi = shi_ref[...]
        dv = dv_ref[...]
        agg = jnp.concatenate(
            [slo[0] + slo[1] + zlo_ref[...], shi[0] + shi[1] + zhi_ref[...]], axis=1)
        pre = agg * dv + b_ref[...]
        h = jnp.maximum(_layer_norm(pre, g_ref[...], be_ref[...]), 0.0)
        m = jnp.maximum(jnp.dot(h, w1_ref[...], preferred_element_type=jnp.float32) + c1_ref[...], 0.0)
        m = jnp.maximum(jnp.dot(m, w2_ref[...], preferred_element_type=jnp.float32) + c2_ref[...], 0.0)
        out_ref[...] = jnp.dot(m, w3_ref[...], preferred_element_type=jnp.float32) + c3_ref[...]

    return pl.pallas_call(
        body,
        grid=(GRID,),
        in_specs=[
            _part_spec(64), _part_spec(64), _row_spec(64), _row_spec(64), _row_spec(1),
            _full_spec((1, 128)), _full_spec((1, 128)), _full_spec((1, 128)),
            _full_spec((128, 128)), _full_spec((1, 128)),
            _full_spec((128, 32)), _full_spec((1, 32)),
            _full_spec((32, 1)), _full_spec((1, 1)),
        ],
        out_specs=_row_spec(1),
        out_shape=jax.ShapeDtypeStruct((N, 1), jnp.float32),
    )(s2lo, s2hi, z2lo, z2hi, dinv, b2, g2, beta2, Wm1, bm1, Wm2, bm2, Wm3, bm3)


def kernel(x, edge_index, W0, b0, g0, beta0, W1, b1, g1, beta1,
           W2, b2, g2, beta2, Wm1, bm1, Wm2, bm2, Wm3, bm3):
    pad = EP - E
    src_p = jnp.concatenate([edge_index[0], jnp.zeros((pad,), edge_index.dtype)])
    dst_p = jnp.concatenate([edge_index[1], jnp.full((pad,), N, edge_index.dtype)])
    src3d = src_p.reshape(NW, NCH, CH)
    dst3d = dst_p.reshape(NW, NCH, CH)

    degp = _deg_kernel(dst3d)
    z0p, dinv = _tc_a(x, W0, degp)
    s0 = _agg64(z0p, src3d, dst3d)
    z1p = _tc_b0(s0, z0p, dinv, b0.reshape(1, -1), g0.reshape(1, -1), beta0.reshape(1, -1))
    s1 = _agg64(z1p, src3d, dst3d)
    z2lo, z2hi = _tc_b1(s1, z1p, dinv, W1, b1.reshape(1, -1), g1.reshape(1, -1),
                        beta1.reshape(1, -1), W2)
    s2lo, s2hi = _agg64x2(z2lo, z2hi, src3d, dst3d)
    y = _tc_b2(s2lo, s2hi, z2lo, z2hi, dinv,
               b2.reshape(1, -1), g2.reshape(1, -1), beta2.reshape(1, -1),
               Wm1, bm1.reshape(1, -1), Wm2, bm2.reshape(1, -1), Wm3, bm3.reshape(1, -1))
    return y


# speedup vs baseline: 1.0004x; 1.0004x over previous
import functools

import jax
import jax.numpy as jnp
from jax import lax
from jax.experimental import pallas as pl
from jax.experimental.pallas import tpu as pltpu
from jax.experimental.pallas import tpu_sc as plsc

N = 10000
E = 320000
IN_DIM = 128
EPS = 1e-5

NC, NS = 2, 16
NW = NC * NS
CH = 128
NCH = 80
EP = NW * NCH * CH
NPAD = 10240
RPT = NPAD // NS
ZB = 128

_MESH = plsc.VectorSubcoreMesh(
    core_axis_name="c", subcore_axis_name="s", num_cores=NC, num_subcores=NS
)


def _fill(buf, nrows, width, value):
    @pl.loop(0, nrows)
    def _(i):
        for j in range(width // 16):
            buf[i, pl.ds(j * 16, 16)] = jnp.full((16,), value, jnp.float32)


def _zero_acc_rows(zbuf, acc, r0, zb=ZB):
    for k in range(RPT // zb):
        pltpu.sync_copy(zbuf, acc.at[pl.ds(r0 + k * zb, zb)])


def _dump_acc(acc, out_hbm, c, r0):
    @pl.when(c == 0)
    def _():
        pltpu.sync_copy(acc.at[pl.ds(r0, RPT)], out_hbm.at[0, pl.ds(r0, RPT)])

    @pl.when(c == 1)
    def _():
        pltpu.sync_copy(acc.at[pl.ds(r0, RPT)], out_hbm.at[1, pl.ds(r0, RPT)])


D64 = 64
MBUF = 3
LOOK = 2
NML = NCH - 2


def _agg_pass(zp_hbm, out_hbm, acc, zps, src_all, dst_all, b0, b1, b2,
              gsem, ssem, c, s, r0):
    bufs = (b0, b1, b2)
    _fill(bufs[0], CH, D64, 0.0)
    _zero_acc_rows(bufs[0], acc, r0, CH)
    @pl.when(s < NS - 1)
    def _():
        pltpu.sync_copy(zp_hbm.at[pl.ds(r0, RPT)], zps.at[pl.ds(r0, RPT)])

    @pl.when(s == NS - 1)
    def _():
        pltpu.sync_copy(zp_hbm.at[pl.ds((NS - 1) * RPT, N - (NS - 1) * RPT)],
                        zps.at[pl.ds((NS - 1) * RPT, N - (NS - 1) * RPT)])

    plsc.subcore_barrier()

    for m in range(LOOK):
        pltpu.async_copy(zps.at[src_all.at[m]], bufs[m], gsem)

    @pl.loop(0, NML, step=MBUF)
    def _(j):
        for m in range(MBUF):
            jj = j + m
            b = bufs[m]
            bn = bufs[(m + LOOK) % MBUF]
            pltpu.make_async_copy(zps.at[src_all.at[jj]], b, gsem).wait()
            pltpu.async_copy(b, acc.at[dst_all.at[jj]], ssem, add=True)

            @pl.when(jj >= 1)
            def _():
                pltpu.make_async_copy(bn, acc.at[dst_all.at[jj - 1]], ssem).wait()

            @pl.when(jj + LOOK < NCH)
            def _():
                pltpu.async_copy(zps.at[src_all.at[jj + LOOK]], bn, gsem)

    for jj in (NML, NML + 1):
        b = bufs[jj % MBUF]
        pltpu.make_async_copy(zps.at[src_all.at[jj]], b, gsem).wait()
        pltpu.async_copy(b, acc.at[dst_all.at[jj]], ssem, add=True)
    for jj in (NML - 1, NML, NML + 1):
        pltpu.make_async_copy(bufs[jj % MBUF], acc.at[dst_all.at[jj]], ssem).wait()

    plsc.subcore_barrier()
    _dump_acc(acc, out_hbm, c, r0)


_AGG_SCRATCH = [
    pltpu.VMEM_SHARED((NPAD, D64), jnp.float32),
    pltpu.VMEM_SHARED((NPAD, D64), jnp.float32),
    pltpu.VMEM((NCH, CH), jnp.int32),
    pltpu.VMEM((NCH, CH), jnp.int32),
    [pltpu.VMEM((CH, D64), jnp.float32) for _ in range(MBUF)],
    pltpu.SemaphoreType.DMA,
    pltpu.SemaphoreType.DMA,
    pltpu.SemaphoreType.DMA,
]


def _agg_prelude(src_hbm, dst_hbm, src_all, dst_all, isem):
    c = lax.axis_index("c")
    s = lax.axis_index("s")
    wid = s * NC + c
    r0 = s * RPT
    pltpu.async_copy(src_hbm.at[wid], src_all, isem)
    pltpu.async_copy(dst_hbm.at[wid], dst_all, isem)
    pltpu.make_async_copy(src_hbm.at[wid], src_all, isem).wait()
    pltpu.make_async_copy(dst_hbm.at[wid], dst_all, isem).wait()
    return c, s, r0


@functools.partial(
    pl.kernel,
    out_type=jax.ShapeDtypeStruct((NC, NPAD, D64), jnp.float32),
    mesh=_MESH,
    scratch_types=_AGG_SCRATCH,
    compiler_params=pltpu.CompilerParams(use_tc_tiling_on_sc=False),
)
def _agg64(zp_hbm, src_hbm, dst_hbm, out_hbm, acc, zps, src_all, dst_all,
           bufs, isem, gsem, ssem):
    c, s, r0 = _agg_prelude(src_hbm, dst_hbm, src_all, dst_all, isem)
    _agg_pass(zp_hbm, out_hbm, acc, zps, src_all, dst_all, bufs[0], bufs[1], bufs[2],
              gsem, ssem, c, s, r0)


DW = 16


@functools.partial(
    pl.kernel,
    out_type=jax.ShapeDtypeStruct((NC, NPAD, DW), jnp.float32),
    mesh=_MESH,
    scratch_types=[
        pltpu.VMEM_SHARED((NPAD, DW), jnp.float32),
        pltpu.VMEM((NCH, CH), jnp.int32),
        pltpu.VMEM((ZB, DW), jnp.float32),
        pltpu.VMEM((CH, DW), jnp.float32),
    ],
    compiler_params=pltpu.CompilerParams(use_tc_tiling_on_sc=False),
)
def _deg_kernel(dst_hbm, out_hbm, acc, dst_all, zbuf, ones_v):
    c = lax.axis_index("c")
    s = lax.axis_index("s")
    wid = s * NC + c
    r0 = s * RPT
    _fill(zbuf, ZB, DW, 0.0)
    _fill(ones_v, CH, DW, 1.0)
    _zero_acc_rows(zbuf, acc, r0)
    pltpu.sync_copy(dst_hbm.at[wid], dst_all)
    plsc.subcore_barrier()

    @pl.loop(0, NCH)
    def _(j):
        pltpu.sync_copy(ones_v, acc.at[dst_all.at[j]], add=True)

    plsc.subcore_barrier()
    _dump_acc(acc, out_hbm, c, r0)


BLK = 512
GRID = (N + BLK - 1) // BLK


def _row_spec(d):
    return pl.BlockSpec((BLK, d), lambda i: (i, 0))


def _full_spec(shape):
    nd = len(shape)
    return pl.BlockSpec(shape, lambda i: (0,) * nd)


def _part_spec(d):
    return pl.BlockSpec((2, BLK, d), lambda i: (0, i, 0))


def _layer_norm(pre, g, b):
    mu = jnp.mean(pre, axis=-1, keepdims=True)
    var = jnp.mean((pre - mu) ** 2, axis=-1, keepdims=True)
    return (pre - mu) / jnp.sqrt(var + EPS) * g + b


def _tc_a(x, W0, degp):
    def body(x_ref, w_ref, dp_ref, z_ref, dv_ref):
        dp = dp_ref[...]
        deg = dp[0, :, 0] + dp[1, :, 0] + 1.0
        dv = lax.rsqrt(deg)[:, None]
        dv_ref[...] = dv
        z_ref[...] = jnp.dot(x_ref[...], w_ref[...], preferred_element_type=jnp.float32) * dv

    return pl.pallas_call(
        body,
        grid=(GRID,),
        in_specs=[_row_spec(IN_DIM), _full_spec((IN_DIM, 64)), _part_spec(DW)],
        out_specs=[_row_spec(64), _row_spec(1)],
        out_shape=[
            jax.ShapeDtypeStruct((N, 64), jnp.float32),
            jax.ShapeDtypeStruct((N, 1), jnp.float32),
        ],
    )(x, W0, degp)


def _tc_b0(s0, z0p, dinv, b0, g0, beta0):
    def body(sp_ref, zp_ref, dv_ref, b_ref, g_ref, be_ref, out_ref):
        sp = sp_ref[...]
        dv = dv_ref[...]
        pre = (sp[0] + sp[1] + zp_ref[...]) * dv + b_ref[...]
        h = jnp.maximum(_layer_norm(pre, g_ref[...], be_ref[...]), 0.0)
        out_ref[...] = h * dv

    return pl.pallas_call(
        body,
        grid=(GRID,),
        in_specs=[
            _part_spec(64), _row_spec(64), _row_spec(1),
            _full_spec((1, 64)), _full_spec((1, 64)), _full_spec((1, 64)),
        ],
        out_specs=_row_spec(64),
        out_shape=jax.ShapeDtypeStruct((N, 64), jnp.float32),
    )(s0, z0p, dinv, b0, g0, beta0)


def _tc_b1(s1, z1p, dinv, W1, b1, g1, beta1, W2):
    def body(sp_ref, zp_ref, dv_ref, w1_ref, b_ref, g_ref, be_ref, w2_ref,
             lo_ref, hi_ref):
        sp = sp_ref[...]
        dv = dv_ref[...]
        agg = (sp[0] + sp[1] + zp_ref[...]) * dv
        pre = jnp.dot(agg, w1_ref[...], preferred_element_type=jnp.float32) + b_ref[...]
        h = jnp.maximum(_layer_norm(pre, g_ref[...], be_ref[...]), 0.0)
        z2 = jnp.dot(h, w2_ref[...], preferred_element_type=jnp.float32) * dv
        lo_ref[...] = z2[:, :64]
        hi_ref[...] = z2[:, 64:]

    return pl.pallas_call(
        body,
        grid=(GRID,),
        in_specs=[
            _part_spec(64), _row_spec(64), _row_spec(1),
            _full_spec((64, 128)), _full_spec((1, 128)), _full_spec((1, 128)),
            _full_spec((1, 128)), _full_spec((128, 128)),
        ],
        out_specs=[_row_spec(64), _row_spec(64)],
        out_shape=[
            jax.ShapeDtypeStruct((N, 64), jnp.float32),
            jax.ShapeDtypeStruct((N, 64), jnp.float32),
        ],
    )(s1, z1p, dinv, W1, b1, g1, beta1, W2)


def _tc_b2(s2lo, s2hi, z2lo, z2hi, dinv, b2, g2, beta2, Wm1, bm1, Wm2, bm2, Wm3, bm3):
    def body(slo_ref, shi_ref, zlo_ref, zhi_ref, dv_ref, b_ref, g_ref, be_ref,
             w1_ref, c1_ref, w2_ref, c2_ref, w3_ref, c3_ref, out_ref):
        slo = slo_ref[...]
        shi = shi_ref[...]
        dv = dv_ref[...]
        agg = jnp.concatenate(
            [slo[0] + slo[1] + zlo_ref[...], shi[0] + shi[1] + zhi_ref[...]], axis=1)
        pre = agg * dv + b_ref[...]
        h = jnp.maximum(_layer_norm(pre, g_ref[...], be_ref[...]), 0.0)
        m = jnp.maximum(jnp.dot(h, w1_ref[...], preferred_element_type=jnp.float32) + c1_ref[...], 0.0)
        m = jnp.maximum(jnp.dot(m, w2_ref[...], preferred_element_type=jnp.float32) + c2_ref[...], 0.0)
        out_ref[...] = jnp.dot(m, w3_ref[...], preferred_element_type=jnp.float32) + c3_ref[...]

    return pl.pallas_call(
        body,
        grid=(GRID,),
        in_specs=[
            _part_spec(64), _part_spec(64), _row_spec(64), _row_spec(64), _row_spec(1),
            _full_spec((1, 128)), _full_spec((1, 128)), _full_spec((1, 128)),
            _full_spec((128, 128)), _full_spec((1, 128)),
            _full_spec((128, 32)), _full_spec((1, 32)),
            _full_spec((32, 1)), _full_spec((1, 1)),
        ],
        out_specs=_row_spec(1),
        out_shape=jax.ShapeDtypeStruct((N, 1), jnp.float32),
    )(s2lo, s2hi, z2lo, z2hi, dinv, b2, g2, beta2, Wm1, bm1, Wm2, bm2, Wm3, bm3)


def kernel(x, edge_index, W0, b0, g0, beta0, W1, b1, g1, beta1,
           W2, b2, g2, beta2, Wm1, bm1, Wm2, bm2, Wm3, bm3):
    pad = EP - E
    src_p = jnp.concatenate([edge_index[0], jnp.zeros((pad,), edge_index.dtype)])
    dst_p = jnp.concatenate([edge_index[1], jnp.full((pad,), N, edge_index.dtype)])
    src3d = src_p.reshape(NW, NCH, CH)
    dst3d = dst_p.reshape(NW, NCH, CH)

    degp = _deg_kernel(dst3d)
    z0p, dinv = _tc_a(x, W0, degp)
    s0 = _agg64(z0p, src3d, dst3d)
    z1p = _tc_b0(s0, z0p, dinv, b0.reshape(1, -1), g0.reshape(1, -1), beta0.reshape(1, -1))
    s1 = _agg64(z1p, src3d, dst3d)
    z2lo, z2hi = _tc_b1(s1, z1p, dinv, W1, b1.reshape(1, -1), g1.reshape(1, -1),
                        beta1.reshape(1, -1), W2)
    s2lo = _agg64(z2lo, src3d, dst3d)
    s2hi = _agg64(z2hi, src3d, dst3d)
    y = _tc_b2(s2lo, s2hi, z2lo, z2hi, dinv,
               b2.reshape(1, -1), g2.reshape(1, -1), beta2.reshape(1, -1),
               Wm1, bm1.reshape(1, -1), Wm2, bm2.reshape(1, -1), Wm3, bm3.reshape(1, -1))
    return y


# speedup vs baseline: 1.0134x; 1.0131x over previous
import functools

import jax
import jax.numpy as jnp
from jax import lax
from jax.experimental import pallas as pl
from jax.experimental.pallas import tpu as pltpu
from jax.experimental.pallas import tpu_sc as plsc

N = 10000
E = 320000
IN_DIM = 128
EPS = 1e-5

NC, NS = 2, 16
NW = NC * NS
CH = 128
NCH = 80
EP = NW * NCH * CH
NPAD = 10240
RPT = NPAD // NS
ZB = 128

_MESH = plsc.VectorSubcoreMesh(
    core_axis_name="c", subcore_axis_name="s", num_cores=NC, num_subcores=NS
)


def _fill(buf, nrows, width, value):
    @pl.loop(0, nrows)
    def _(i):
        for j in range(width // 16):
            buf[i, pl.ds(j * 16, 16)] = jnp.full((16,), value, jnp.float32)


def _zero_acc_rows(zbuf, acc, r0, zb=ZB):
    for k in range(RPT // zb):
        pltpu.sync_copy(zbuf, acc.at[pl.ds(r0 + k * zb, zb)])


def _dump_acc(acc, out_hbm, c, r0):
    @pl.when(c == 0)
    def _():
        pltpu.sync_copy(acc.at[pl.ds(r0, RPT)], out_hbm.at[0, pl.ds(r0, RPT)])

    @pl.when(c == 1)
    def _():
        pltpu.sync_copy(acc.at[pl.ds(r0, RPT)], out_hbm.at[1, pl.ds(r0, RPT)])


D64 = 64
MBUF = 3
LOOK = 2
NML = NCH - 2


def _agg_pass(zp_hbm, out_hbm, acc, zps, src_all, dst_all, b0, b1, b2,
              gsem, ssem, c, s, r0, wait_idx=None):
    bufs = (b0, b1, b2)
    _fill(bufs[0], CH, D64, 0.0)
    _zero_acc_rows(bufs[0], acc, r0, CH)
    @pl.when(s < NS - 1)
    def _():
        pltpu.sync_copy(zp_hbm.at[pl.ds(r0, RPT)], zps.at[pl.ds(r0, RPT)])

    @pl.when(s == NS - 1)
    def _():
        pltpu.sync_copy(zp_hbm.at[pl.ds((NS - 1) * RPT, N - (NS - 1) * RPT)],
                        zps.at[pl.ds((NS - 1) * RPT, N - (NS - 1) * RPT)])

    if wait_idx is not None:
        wait_idx()
    plsc.subcore_barrier()

    for m in range(LOOK):
        pltpu.async_copy(zps.at[src_all.at[m]], bufs[m], gsem)

    @pl.loop(0, NML, step=MBUF)
    def _(j):
        for m in range(MBUF):
            jj = j + m
            b = bufs[m]
            bn = bufs[(m + LOOK) % MBUF]
            pltpu.make_async_copy(zps.at[src_all.at[jj]], b, gsem).wait()
            pltpu.async_copy(b, acc.at[dst_all.at[jj]], ssem, add=True)

            @pl.when(jj >= 1)
            def _():
                pltpu.make_async_copy(bn, acc.at[dst_all.at[jj - 1]], ssem).wait()

            @pl.when(jj + LOOK < NCH)
            def _():
                pltpu.async_copy(zps.at[src_all.at[jj + LOOK]], bn, gsem)

    for jj in (NML, NML + 1):
        b = bufs[jj % MBUF]
        pltpu.make_async_copy(zps.at[src_all.at[jj]], b, gsem).wait()
        pltpu.async_copy(b, acc.at[dst_all.at[jj]], ssem, add=True)
    for jj in (NML - 1, NML, NML + 1):
        pltpu.make_async_copy(bufs[jj % MBUF], acc.at[dst_all.at[jj]], ssem).wait()

    plsc.subcore_barrier()
    _dump_acc(acc, out_hbm, c, r0)


_AGG_SCRATCH = [
    pltpu.VMEM_SHARED((NPAD, D64), jnp.float32),
    pltpu.VMEM_SHARED((NPAD, D64), jnp.float32),
    pltpu.VMEM((NCH, CH), jnp.int32),
    pltpu.VMEM((NCH, CH), jnp.int32),
    [pltpu.VMEM((CH, D64), jnp.float32) for _ in range(MBUF)],
    pltpu.SemaphoreType.DMA,
    pltpu.SemaphoreType.DMA,
    pltpu.SemaphoreType.DMA,
]


def _agg_prelude(src_hbm, dst_hbm, src_all, dst_all, isem):
    c = lax.axis_index("c")
    s = lax.axis_index("s")
    wid = s * NC + c
    r0 = s * RPT
    pltpu.async_copy(src_hbm.at[wid], src_all, isem)
    pltpu.async_copy(dst_hbm.at[wid], dst_all, isem)

    def wait_idx():
        pltpu.make_async_copy(src_hbm.at[wid], src_all, isem).wait()
        pltpu.make_async_copy(dst_hbm.at[wid], dst_all, isem).wait()

    return c, s, r0, wait_idx


@functools.partial(
    pl.kernel,
    out_type=jax.ShapeDtypeStruct((NC, NPAD, D64), jnp.float32),
    mesh=_MESH,
    scratch_types=_AGG_SCRATCH,
    compiler_params=pltpu.CompilerParams(use_tc_tiling_on_sc=False),
)
def _agg64(zp_hbm, src_hbm, dst_hbm, out_hbm, acc, zps, src_all, dst_all,
           bufs, isem, gsem, ssem):
    c, s, r0, wait_idx = _agg_prelude(src_hbm, dst_hbm, src_all, dst_all, isem)
    _agg_pass(zp_hbm, out_hbm, acc, zps, src_all, dst_all, bufs[0], bufs[1], bufs[2],
              gsem, ssem, c, s, r0, wait_idx)


DW = 16


@functools.partial(
    pl.kernel,
    out_type=jax.ShapeDtypeStruct((NC, NPAD, DW), jnp.float32),
    mesh=_MESH,
    scratch_types=[
        pltpu.VMEM_SHARED((NPAD, DW), jnp.float32),
        pltpu.VMEM((NCH, CH), jnp.int32),
        pltpu.VMEM((ZB, DW), jnp.float32),
        pltpu.VMEM((CH, DW), jnp.float32),
    ],
    compiler_params=pltpu.CompilerParams(use_tc_tiling_on_sc=False),
)
def _deg_kernel(dst_hbm, out_hbm, acc, dst_all, zbuf, ones_v):
    c = lax.axis_index("c")
    s = lax.axis_index("s")
    wid = s * NC + c
    r0 = s * RPT
    _fill(zbuf, ZB, DW, 0.0)
    _fill(ones_v, CH, DW, 1.0)
    _zero_acc_rows(zbuf, acc, r0)
    pltpu.sync_copy(dst_hbm.at[wid], dst_all)
    plsc.subcore_barrier()

    @pl.loop(0, NCH)
    def _(j):
        pltpu.sync_copy(ones_v, acc.at[dst_all.at[j]], add=True)

    plsc.subcore_barrier()
    _dump_acc(acc, out_hbm, c, r0)


BLK = 512
GRID = (N + BLK - 1) // BLK


def _row_spec(d):
    return pl.BlockSpec((BLK, d), lambda i: (i, 0))


def _full_spec(shape):
    nd = len(shape)
    return pl.BlockSpec(shape, lambda i: (0,) * nd)


def _part_spec(d):
    return pl.BlockSpec((2, BLK, d), lambda i: (0, i, 0))


def _layer_norm(pre, g, b):
    mu = jnp.mean(pre, axis=-1, keepdims=True)
    var = jnp.mean((pre - mu) ** 2, axis=-1, keepdims=True)
    return (pre - mu) / jnp.sqrt(var + EPS) * g + b


def _tc_a(x, W0, degp):
    def body(x_ref, w_ref, dp_ref, z_ref, dv_ref):
        dp = dp_ref[...]
        deg = dp[0, :, 0] + dp[1, :, 0] + 1.0
        dv = lax.rsqrt(deg)[:, None]
        dv_ref[...] = dv
        z_ref[...] = jnp.dot(x_ref[...], w_ref[...], preferred_element_type=jnp.float32) * dv

    return pl.pallas_call(
        body,
        grid=(GRID,),
        in_specs=[_row_spec(IN_DIM), _full_spec((IN_DIM, 64)), _part_spec(DW)],
        out_specs=[_row_spec(64), _row_spec(1)],
        out_shape=[
            jax.ShapeDtypeStruct((N, 64), jnp.float32),
            jax.ShapeDtypeStruct((N, 1), jnp.float32),
        ],
    )(x, W0, degp)


def _tc_b0(s0, z0p, dinv, b0, g0, beta0):
    def body(sp_ref, zp_ref, dv_ref, b_ref, g_ref, be_ref, out_ref):
        sp = sp_ref[...]
        dv = dv_ref[...]
        pre = (sp[0] + sp[1] + zp_ref[...]) * dv + b_ref[...]
        h = jnp.maximum(_layer_norm(pre, g_ref[...], be_ref[...]), 0.0)
        out_ref[...] = h * dv

    return pl.pallas_call(
        body,
        grid=(GRID,),
        in_specs=[
            _part_spec(64), _row_spec(64), _row_spec(1),
            _full_spec((1, 64)), _full_spec((1, 64)), _full_spec((1, 64)),
        ],
        out_specs=_row_spec(64),
        out_shape=jax.ShapeDtypeStruct((N, 64), jnp.float32),
    )(s0, z0p, dinv, b0, g0, beta0)


def _tc_b1(s1, z1p, dinv, W1, b1, g1, beta1, W2):
    def body(sp_ref, zp_ref, dv_ref, w1_ref, b_ref, g_ref, be_ref, w2_ref,
             lo_ref, hi_ref):
        sp = sp_ref[...]
        dv = dv_ref[...]
        agg = (sp[0] + sp[1] + zp_ref[...]) * dv
        pre = jnp.dot(agg, w1_ref[...], preferred_element_type=jnp.float32) + b_ref[...]
        h = jnp.maximum(_layer_norm(pre, g_ref[...], be_ref[...]), 0.0)
        z2 = jnp.dot(h, w2_ref[...], preferred_element_type=jnp.float32) * dv
        lo_ref[...] = z2[:, :64]
        hi_ref[...] = z2[:, 64:]

    return pl.pallas_call(
        body,
        grid=(GRID,),
        in_specs=[
            _part_spec(64), _row_spec(64), _row_spec(1),
            _full_spec((64, 128)), _full_spec((1, 128)), _full_spec((1, 128)),
            _full_spec((1, 128)), _full_spec((128, 128)),
        ],
        out_specs=[_row_spec(64), _row_spec(64)],
        out_shape=[
            jax.ShapeDtypeStruct((N, 64), jnp.float32),
            jax.ShapeDtypeStruct((N, 64), jnp.float32),
        ],
    )(s1, z1p, dinv, W1, b1, g1, beta1, W2)


def _tc_b2(s2lo, s2hi, z2lo, z2hi, dinv, b2, g2, beta2, Wm1, bm1, Wm2, bm2, Wm3, bm3):
    def body(slo_ref, shi_ref, zlo_ref, zhi_ref, dv_ref, b_ref, g_ref, be_ref,
             w1_ref, c1_ref, w2_ref, c2_ref, w3_ref, c3_ref, out_ref):
        slo = slo_ref[...]
        shi = shi_ref[...]
        dv = dv_ref[...]
        agg = jnp.concatenate(
            [slo[0] + slo[1] + zlo_ref[...], shi[0] + shi[1] + zhi_ref[...]], axis=1)
        pre = agg * dv + b_ref[...]
        h = jnp.maximum(_layer_norm(pre, g_ref[...], be_ref[...]), 0.0)
        m = jnp.maximum(jnp.dot(h, w1_ref[...], preferred_element_type=jnp.float32) + c1_ref[...], 0.0)
        m = jnp.maximum(jnp.dot(m, w2_ref[...], preferred_element_type=jnp.float32) + c2_ref[...], 0.0)
        out_ref[...] = jnp.dot(m, w3_ref[...], preferred_element_type=jnp.float32) + c3_ref[...]

    return pl.pallas_call(
        body,
        grid=(GRID,),
        in_specs=[
            _part_spec(64), _part_spec(64), _row_spec(64), _row_spec(64), _row_spec(1),
            _full_spec((1, 128)), _full_spec((1, 128)), _full_spec((1, 128)),
            _full_spec((128, 128)), _full_spec((1, 128)),
            _full_spec((128, 32)), _full_spec((1, 32)),
            _full_spec((32, 1)), _full_spec((1, 1)),
        ],
        out_specs=_row_spec(1),
        out_shape=jax.ShapeDtypeStruct((N, 1), jnp.float32),
    )(s2lo, s2hi, z2lo, z2hi, dinv, b2, g2, beta2, Wm1, bm1, Wm2, bm2, Wm3, bm3)


def kernel(x, edge_index, W0, b0, g0, beta0, W1, b1, g1, beta1,
           W2, b2, g2, beta2, Wm1, bm1, Wm2, bm2, Wm3, bm3):
    pad = EP - E
    src_p = jnp.concatenate([edge_index[0], jnp.zeros((pad,), edge_index.dtype)])
    dst_p = jnp.concatenate([edge_index[1], jnp.full((pad,), N, edge_index.dtype)])
    src3d = src_p.reshape(NW, NCH, CH)
    dst3d = dst_p.reshape(NW, NCH, CH)

    degp = _deg_kernel(dst3d)
    z0p, dinv = _tc_a(x, W0, degp)
    s0 = _agg64(z0p, src3d, dst3d)
    z1p = _tc_b0(s0, z0p, dinv, b0.reshape(1, -1), g0.reshape(1, -1), beta0.reshape(1, -1))
    s1 = _agg64(z1p, src3d, dst3d)
    z2lo, z2hi = _tc_b1(s1, z1p, dinv, W1, b1.reshape(1, -1), g1.reshape(1, -1),
                        beta1.reshape(1, -1), W2)
    s2lo = _agg64(z2lo, src3d, dst3d)
    s2hi = _agg64(z2hi, src3d, dst3d)
    y = _tc_b2(s2lo, s2hi, z2lo, z2hi, dinv,
               b2.reshape(1, -1), g2.reshape(1, -1), beta2.reshape(1, -1),
               Wm1, bm1.reshape(1, -1), Wm2, bm2.reshape(1, -1), Wm3, bm3.reshape(1, -1))
    return y
